# baseline (speedup 1.0000x reference)
import functools

import jax
import jax.numpy as jnp
from jax import lax
from jax.experimental import pallas as pl
from jax.experimental.pallas import tpu as pltpu
from jax.experimental.pallas import tpu_sc as plsc

VOCAB = 100000
D = 16
B = 1024
CTX = 20

NC = 2
NS = 16
NW = NC * NS
BPW = B // NW
IDX_PER_W = BPW * CTX
CHUNK = 128
NCHUNK = IDX_PER_W // CHUNK


def _pool_body(ctx_hbm, table_hbm, out_hbm, idx_v, rows_v, pooled_v, sem):
    wid = lax.axis_index("s") * NC + lax.axis_index("c")
    base = wid * BPW
    pltpu.sync_copy(ctx_hbm.at[wid], idx_v)
    copies = []
    for j in range(NCHUNK):
        copies.append(
            pltpu.async_copy(
                table_hbm.at[idx_v.at[j]],
                rows_v.at[pl.ds(j * CHUNK, CHUNK)],
                sem,
            )
        )
    for c in copies:
        c.wait()
    scale = jnp.float32(1.0 / CTX)
    for r in range(BPW):
        acc = rows_v[r * CTX, :]
        for t in range(1, CTX):
            acc = acc + rows_v[r * CTX + t, :]
        pooled_v[r, :] = acc * scale
    pltpu.sync_copy(pooled_v, out_hbm.at[pl.ds(base, BPW)])


@functools.cache
def _pool():
    return pl.kernel(
        _pool_body,
        out_type=jax.ShapeDtypeStruct((B, D), jnp.float32),
        mesh=plsc.VectorSubcoreMesh(
            core_axis_name="c", subcore_axis_name="s",
            num_cores=NC, num_subcores=NS,
        ),
        scratch_types=[
            pltpu.VMEM((NCHUNK, CHUNK), jnp.int32),
            pltpu.VMEM((IDX_PER_W, D), jnp.float32),
            pltpu.VMEM((BPW, D), jnp.float32),
            pltpu.SemaphoreType.DMA,
        ],
        compiler_params=pltpu.CompilerParams(use_tc_tiling_on_sc=False),
    )


TV = 512
NSTEP = pl.cdiv(VOCAB, TV)
LAST = VOCAB - (NSTEP - 1) * TV
NBUF = 6


def _mm_body(w_ref, mean_ref, b_ref, out_hbm, buf, sems):
    j = pl.program_id(0)
    slot = lax.rem(j, NBUF)

    def _desc(step, s, rows):
        return pltpu.make_async_copy(
            buf.at[s, pl.ds(0, rows), :],
            out_hbm.at[pl.ds(step * TV, rows), :],
            sems.at[s],
        )

    @pl.when(j >= NBUF)
    def _():
        _desc(j - NBUF, slot, TV).wait()

    acc = lax.dot_general(
        w_ref[...], mean_ref[...],
        (((1,), (1,)), ((), ())),
        preferred_element_type=jnp.float32,
    )
    buf[slot] = acc + b_ref[...]

    @pl.when(j < NSTEP - 1)
    def _():
        _desc(j, slot, TV).start()

    @pl.when(j == NSTEP - 1)
    def _():
        _desc(j, slot, LAST).start()
        for step in range(NSTEP - NBUF, NSTEP):
            _desc(step, step % NBUF, TV if step < NSTEP - 1 else LAST).wait()


def _project_t(lin_w, mean, lin_b2d):
    return pl.pallas_call(
        _mm_body,
        grid=(NSTEP,),
        in_specs=[
            pl.BlockSpec((TV, D), lambda j: (j, 0)),
            pl.BlockSpec((B, D), lambda j: (0, 0)),
            pl.BlockSpec((TV, 1), lambda j: (j, 0)),
        ],
        out_specs=pl.BlockSpec(memory_space=pl.ANY),
        out_shape=jax.ShapeDtypeStruct((VOCAB, B), jnp.float32),
        scratch_shapes=[
            pltpu.VMEM((NBUF, TV, B), jnp.float32),
            pltpu.SemaphoreType.DMA((NBUF,)),
        ],
    )(lin_w, mean, lin_b2d)


def kernel(context, emb_table, lin_w, lin_b):
    ctx = context.astype(jnp.int32).reshape(NW, NCHUNK, CHUNK)
    mean = _pool()(ctx, emb_table)
    return _project_t(lin_w, mean, lin_b.reshape(VOCAB, 1)).T


# speedup vs baseline: 1.1905x; 1.1905x over previous
import functools

import jax
import jax.numpy as jnp
from jax import lax
from jax.experimental import pallas as pl
from jax.experimental.pallas import tpu as pltpu
from jax.experimental.pallas import tpu_sc as plsc

VOCAB = 100000
D = 16
B = 1024
CTX = 20

NC = 2
NS = 16
NW = NC * NS
BPW = B // NW
IDX_PER_W = BPW * CTX
CHUNK = 128
NCHUNK = IDX_PER_W // CHUNK


def _pool_body(ctx_hbm, table_hbm, out_hbm, idx_v, rows_v, pooled_v, sem):
    wid = lax.axis_index("s") * NC + lax.axis_index("c")
    base = wid * BPW
    pltpu.sync_copy(ctx_hbm.at[wid], idx_v)
    copies = []
    for j in range(NCHUNK):
        copies.append(
            pltpu.async_copy(
                table_hbm.at[idx_v.at[j]],
                rows_v.at[pl.ds(j * CHUNK, CHUNK)],
                sem,
            )
        )
    for c in copies:
        c.wait()
    scale = jnp.float32(1.0 / CTX)
    for r in range(BPW):
        acc = rows_v[r * CTX, :]
        for t in range(1, CTX):
            acc = acc + rows_v[r * CTX + t, :]
        pooled_v[r, :] = acc * scale
    pltpu.sync_copy(pooled_v, out_hbm.at[pl.ds(base, BPW)])


@functools.cache
def _pool():
    return pl.kernel(
        _pool_body,
        out_type=jax.ShapeDtypeStruct((B, D), jnp.float32),
        mesh=plsc.VectorSubcoreMesh(
            core_axis_name="c", subcore_axis_name="s",
            num_cores=NC, num_subcores=NS,
        ),
        scratch_types=[
            pltpu.VMEM((NCHUNK, CHUNK), jnp.int32),
            pltpu.VMEM((IDX_PER_W, D), jnp.float32),
            pltpu.VMEM((BPW, D), jnp.float32),
            pltpu.SemaphoreType.DMA,
        ],
        compiler_params=pltpu.CompilerParams(use_tc_tiling_on_sc=False),
    )


TV = 2048


def _mm_body(w_ref, mean_ref, b_ref, out_ref):
    acc = lax.dot_general(
        w_ref[...], mean_ref[...],
        (((1,), (1,)), ((), ())),
        preferred_element_type=jnp.float32,
    )
    out_ref[...] = acc + b_ref[...]


def _project_t(lin_w, mean, lin_b2d):
    grid = (pl.cdiv(VOCAB, TV),)
    return pl.pallas_call(
        _mm_body,
        grid=grid,
        in_specs=[
            pl.BlockSpec((TV, D), lambda j: (j, 0)),
            pl.BlockSpec((B, D), lambda j: (0, 0)),
            pl.BlockSpec((TV, 1), lambda j: (j, 0)),
        ],
        out_specs=pl.BlockSpec((TV, B), lambda j: (j, 0)),
        out_shape=jax.ShapeDtypeStruct((VOCAB, B), jnp.float32),
    )(lin_w, mean, lin_b2d)


def kernel(context, emb_table, lin_w, lin_b):
    ctx = context.astype(jnp.int32).reshape(NW, NCHUNK, CHUNK)
    mean = _pool()(ctx, emb_table)
    return _project_t(lin_w, mean, lin_b.reshape(VOCAB, 1)).T


# speedup vs baseline: 1.2264x; 1.0302x over previous
import functools

import jax
import jax.numpy as jnp
from jax import lax
from jax.experimental import pallas as pl
from jax.experimental.pallas import tpu as pltpu
from jax.experimental.pallas import tpu_sc as plsc

VOCAB = 100000
D = 16
B = 1024
CTX = 20

NC = 2
NS = 16
NW = NC * NS
BPW = B // NW
IDX_PER_W = BPW * CTX
CHUNK = 128
NCHUNK = IDX_PER_W // CHUNK


def _pool_body(ctx_hbm, table_hbm, out_hbm, idx_v, rows_v, pooled_v, sem):
    wid = lax.axis_index("s") * NC + lax.axis_index("c")
    base = wid * BPW
    pltpu.sync_copy(ctx_hbm.at[wid], idx_v)
    copies = []
    for j in range(NCHUNK):
        copies.append(
            pltpu.async_copy(
                table_hbm.at[idx_v.at[j]],
                rows_v.at[pl.ds(j * CHUNK, CHUNK)],
                sem,
            )
        )
    for c in copies:
        c.wait()
    scale = jnp.float32(1.0 / CTX)
    for r in range(BPW):
        acc = rows_v[r * CTX, :]
        for t in range(1, CTX):
            acc = acc + rows_v[r * CTX + t, :]
        pooled_v[r, :] = acc * scale
    pltpu.sync_copy(pooled_v, out_hbm.at[pl.ds(base, BPW)])


@functools.cache
def _pool():
    return pl.kernel(
        _pool_body,
        out_type=jax.ShapeDtypeStruct((B, D), jnp.float32),
        mesh=plsc.VectorSubcoreMesh(
            core_axis_name="c", subcore_axis_name="s",
            num_cores=NC, num_subcores=NS,
        ),
        scratch_types=[
            pltpu.VMEM((NCHUNK, CHUNK), jnp.int32),
            pltpu.VMEM((IDX_PER_W, D), jnp.float32),
            pltpu.VMEM((BPW, D), jnp.float32),
            pltpu.SemaphoreType.DMA,
        ],
        compiler_params=pltpu.CompilerParams(use_tc_tiling_on_sc=False),
    )


TV = 4096


def _mm_body(w_ref, mean_ref, b_ref, out_ref):
    acc = lax.dot_general(
        w_ref[...], mean_ref[...],
        (((1,), (1,)), ((), ())),
        preferred_element_type=jnp.float32,
    )
    out_ref[...] = acc + b_ref[...]


def _project_t(lin_w, mean, lin_b2d):
    grid = (pl.cdiv(VOCAB, TV),)
    return pl.pallas_call(
        _mm_body,
        grid=grid,
        in_specs=[
            pl.BlockSpec((TV, D), lambda j: (j, 0)),
            pl.BlockSpec((B, D), lambda j: (0, 0)),
            pl.BlockSpec((TV, 1), lambda j: (j, 0)),
        ],
        out_specs=pl.BlockSpec((TV, B), lambda j: (j, 0)),
        out_shape=jax.ShapeDtypeStruct((VOCAB, B), jnp.float32),
    )(lin_w, mean, lin_b2d)


def kernel(context, emb_table, lin_w, lin_b):
    ctx = context.astype(jnp.int32).reshape(NW, NCHUNK, CHUNK)
    mean = _pool()(ctx, emb_table)
    return _project_t(lin_w, mean, lin_b.reshape(VOCAB, 1)).T
